# baseline (speedup 1.0000x reference)
import dataclasses
import functools

import jax
import jax.numpy as jnp
from jax import lax
from jax.experimental import pallas as pl
from jax.experimental.pallas import tpu as pltpu
from jax.experimental.pallas import tpu_sc as plsc

NSEG = 1024
ROWS = 50000
FEAT = 256
CHUNK = 2000
NCHUNK = ROWS // CHUNK
NC, NS, LANES = 2, 16, 16
NW = NC * NS
TSLICE = 1568
TSLICE_LAST = ROWS - (NW - 1) * TSLICE

_mesh = plsc.VectorSubcoreMesh(core_axis_name="c", subcore_axis_name="s")

_sc_params = pltpu.CompilerParams()
if "needs_layout_passes" in pltpu.CompilerParams.__dataclass_fields__:
    _sc_params = dataclasses.replace(_sc_params, needs_layout_passes=False)


def _sums_body(b_ref, x_ref, o_ref):
    i = pl.program_id(0)
    bvec = b_ref[0, 0, :]
    gids = jax.lax.broadcasted_iota(jnp.int32, (NSEG, CHUNK), 0)
    onehot = (gids == bvec[None, :]).astype(jnp.bfloat16)
    psum = jax.lax.dot(onehot, x_ref[...].astype(jnp.bfloat16),
                       preferred_element_type=jnp.float32)

    @pl.when(i == 0)
    def _():
        o_ref[...] = psum

    @pl.when(i > 0)
    def _():
        o_ref[...] += psum


def _tc_sums(x, b3):
    return pl.pallas_call(
        _sums_body,
        grid=(NCHUNK,),
        in_specs=[
            pl.BlockSpec((1, 1, CHUNK), lambda i: (i, 0, 0)),
            pl.BlockSpec((CHUNK, FEAT), lambda i: (i, 0)),
        ],
        out_specs=pl.BlockSpec((NSEG, FEAT), lambda i: (0, 0)),
        out_shape=jax.ShapeDtypeStruct((NSEG, FEAT), jnp.float32),
    )(b3, x)


@functools.partial(
    pl.kernel,
    mesh=_mesh,
    out_type=jax.ShapeDtypeStruct((NW, NSEG), jnp.float32),
    compiler_params=_sc_params,
    scratch_types=[
        pltpu.VMEM((TSLICE,), jnp.int32),
        pltpu.VMEM((NSEG,), jnp.float32),
    ],
)
def _sc_counts(b_hbm, cnt_hbm, bbuf, cnt):
    c = lax.axis_index("c")
    s = lax.axis_index("s")
    w = c * NS + s

    zero = jnp.zeros((LANES,), jnp.float32)
    one = jnp.ones((LANES,), jnp.float32)

    @pl.loop(0, NSEG // LANES)
    def _(i):
        cnt[pl.ds(i * LANES, LANES)] = zero

    base = w * TSLICE

    @pl.when(w < NW - 1)
    def _():
        pltpu.sync_copy(b_hbm.at[pl.ds(base, TSLICE)], bbuf)

    @pl.when(w == NW - 1)
    def _():
        pltpu.sync_copy(b_hbm.at[pl.ds(base, TSLICE_LAST)],
                        bbuf.at[pl.ds(0, TSLICE_LAST)])

    nit = jnp.where(w == NW - 1, TSLICE_LAST // LANES, TSLICE // LANES)

    @pl.loop(0, nit)
    def _(i):
        idx = bbuf[pl.ds(i * LANES, LANES)]
        plsc.addupdate_scatter(cnt, [idx], one)

    pltpu.sync_copy(cnt, cnt_hbm.at[w])


def _combine_body(s_ref, c_ref, o_ref):
    cnt = jnp.sum(c_ref[...], axis=0, keepdims=True)
    o_ref[...] = s_ref[...] / jnp.maximum(cnt.reshape(NSEG, 1), 1.0)


def kernel(x, batch):
    b = batch.astype(jnp.int32)
    b3 = b.reshape(NCHUNK, 1, CHUNK)
    sums = _tc_sums(x, b3)
    cnts = _sc_counts(b)
    return pl.pallas_call(
        _combine_body,
        out_shape=jax.ShapeDtypeStruct((NSEG, FEAT), jnp.float32),
    )(sums, cnts)
